# baseline (speedup 1.0000x reference)
import functools

import jax
import jax.numpy as jnp
from jax import lax
from jax.experimental import pallas as pl
from jax.experimental.pallas import tpu as pltpu
from jax.experimental.pallas import tpu_sc as plsc

_N_VOCAB = 1000000
_N_EMBD = 32
_PAD = 33
_CHUNK = 2048

_W = 512
_FULL_CHUNKS = _N_VOCAB // _W
_TAIL = _N_VOCAB - _FULL_CHUNKS * _W


@functools.cache
def _make_detranspose():
    info = plsc.get_sparse_core_info()
    nw = info.num_cores * info.num_subcores
    max_j2 = (-(-_FULL_CHUNKS // nw) + 1) // 2
    mesh = plsc.VectorSubcoreMesh(core_axis_name="c", subcore_axis_name="s")

    @functools.partial(
        pl.kernel,
        mesh=mesh,
        out_type=jax.ShapeDtypeStruct((_N_VOCAB * _PAD,), jnp.float32),
        scratch_types=[
            pltpu.VMEM((_N_EMBD, _W), jnp.float32),
            pltpu.VMEM((_N_EMBD, _W), jnp.float32),
            pltpu.VMEM((_W * _PAD,), jnp.float32),
            pltpu.VMEM((_W * _PAD,), jnp.float32),
            pltpu.VMEM((_N_EMBD, _TAIL), jnp.float32),
            pltpu.VMEM((_TAIL * _PAD,), jnp.float32),
            pltpu.SemaphoreType.DMA,
            pltpu.SemaphoreType.DMA,
            pltpu.SemaphoreType.DMA,
            pltpu.SemaphoreType.DMA,
        ],
        compiler_params=pltpu.CompilerParams(needs_layout_passes=False),
    )
    def k1(tt, out, tin0, tin1, rout0, rout1, tin_t, rout_t,
           isem0, isem1, osem0, osem1):
        wid = lax.axis_index("s") * info.num_cores + lax.axis_index("c")
        tin = (tin0, tin1)
        rout = (rout0, rout1)
        isem = (isem0, isem1)
        osem = (osem0, osem1)
        iotp = lax.iota(jnp.int32, 16) * _PAD
        n_valid = (_FULL_CHUNKS - wid + nw - 1) // nw

        def in_slice(c):
            return tt.at[:, pl.ds(c * _W, _W)]

        def out_slice(c):
            return out.at[pl.ds(c * (_W * _PAD), _W * _PAD)]

        def start_in(j, b):
            c = wid + j * nw

            @pl.when(c < _FULL_CHUNKS)
            def _():
                pltpu.async_copy(in_slice(c), tin[b], isem[b])

        def transpose_chunk(src, dst, ncols):
            def kbody(k, carry):
                base = iotp + k * 16 * _PAD

                def ebody(e, carry2):
                    v = src[e, pl.ds(k * 16, 16)]
                    plsc.store_scatter(dst, [base + e], v)
                    return carry2

                lax.fori_loop(0, _N_EMBD, ebody, 0, unroll=8)
                return carry

            lax.fori_loop(0, ncols // 16, kbody, 0)

        start_in(0, 0)
        start_in(1, 1)

        def body2(j2, carry):
            for b in range(2):
                j = j2 * 2 + b
                c = wid + j * nw

                @pl.when(c < _FULL_CHUNKS)
                def _():
                    @pl.when(j2 >= 1)
                    def _():
                        pltpu.make_async_copy(
                            rout[b], out_slice(c), osem[b]).wait()

                    pltpu.make_async_copy(in_slice(c), tin[b], isem[b]).wait()
                    transpose_chunk(tin[b], rout[b], _W)
                    pltpu.async_copy(rout[b], out_slice(c), osem[b])
                    start_in(j + 2, b)

            return carry

        lax.fori_loop(0, max_j2, body2, 0)

        @pl.when(n_valid >= 1)
        def _drain0():
            pltpu.make_async_copy(rout[0], out_slice(0), osem[0]).wait()

        @pl.when(n_valid >= 2)
        def _drain1():
            pltpu.make_async_copy(rout[1], out_slice(0), osem[1]).wait()

        @pl.when(wid == nw - 1)
        def _tail():
            base = _FULL_CHUNKS * _W
            pltpu.sync_copy(tt.at[:, pl.ds(base, _TAIL)], tin_t)
            transpose_chunk(tin_t, rout_t, _TAIL)
            pltpu.sync_copy(
                rout_t,
                out.at[pl.ds(base * _PAD, _TAIL * _PAD)])

    return k1


@functools.cache
def _make_gather(B, D):
    info = plsc.get_sparse_core_info()
    nw = info.num_cores * info.num_subcores
    b_per_w = B // nw
    n_chunks = b_per_w // _CHUNK
    assert b_per_w % _CHUNK == 0
    mesh = plsc.VectorSubcoreMesh(core_axis_name="c", subcore_axis_name="s")

    @functools.partial(
        pl.kernel,
        mesh=mesh,
        out_type=jax.ShapeDtypeStruct((B, D), jnp.float32),
        scratch_types=[
            pltpu.VMEM((_CHUNK,), jnp.int32),
            pltpu.VMEM((_CHUNK, _PAD), jnp.float32),
            pltpu.SemaphoreType.DMA,
        ],
        compiler_params=pltpu.CompilerParams(use_tc_tiling_on_sc=False),
    )
    def k2(idx_hbm, table_hbm, out_hbm, idx_v, rows_v, sem):
        wid = lax.axis_index("s") * info.num_cores + lax.axis_index("c")
        base = wid * b_per_w

        def body(j, carry):
            off = base + j * _CHUNK
            pltpu.sync_copy(idx_hbm.at[pl.ds(off, _CHUNK)], idx_v)
            pltpu.async_copy(table_hbm.at[idx_v], rows_v, sem).wait()
            pltpu.sync_copy(rows_v.at[:, pl.ds(0, D)],
                            out_hbm.at[pl.ds(off, _CHUNK)])
            return carry

        lax.fori_loop(0, n_chunks, body, 0)

    return k2


def kernel(inputs, wte):
    s0, s1 = inputs.shape
    idx = inputs.reshape(s0 * s1).astype(jnp.int32)
    table_pad = _make_detranspose()(wte.T).reshape(_N_VOCAB, _PAD)
    gathered = _make_gather(s0 * s1, _N_EMBD)(idx, table_pad)
    return (gathered.reshape(s0, s1, _N_EMBD), wte)


# speedup vs baseline: 2.0006x; 2.0006x over previous
import functools

import jax
import jax.numpy as jnp
from jax import lax
from jax.experimental import pallas as pl
from jax.experimental.pallas import tpu as pltpu
from jax.experimental.pallas import tpu_sc as plsc

_N_VOCAB = 1000000
_N_EMBD = 32
_CHUNK = 2048

_W = 512
_WP = _W + 1
_FULL_CHUNKS = _N_VOCAB // _W
_TAIL = _N_VOCAB - _FULL_CHUNKS * _W


@functools.cache
def _make_detranspose():
    info = plsc.get_sparse_core_info()
    nw = info.num_cores * info.num_subcores
    max_j2 = (-(-_FULL_CHUNKS // nw) + 1) // 2
    mesh = plsc.VectorSubcoreMesh(core_axis_name="c", subcore_axis_name="s")

    @functools.partial(
        pl.kernel,
        mesh=mesh,
        out_type=jax.ShapeDtypeStruct((_N_VOCAB * _N_EMBD,), jnp.float32),
        scratch_types=[
            pltpu.VMEM((_N_EMBD, _WP), jnp.float32),
            pltpu.VMEM((_N_EMBD, _WP), jnp.float32),
            pltpu.VMEM((_W * _N_EMBD,), jnp.float32),
            pltpu.VMEM((_W * _N_EMBD,), jnp.float32),
            pltpu.VMEM((_N_EMBD, _TAIL), jnp.float32),
            pltpu.VMEM((_TAIL * _N_EMBD,), jnp.float32),
            pltpu.SemaphoreType.DMA,
            pltpu.SemaphoreType.DMA,
            pltpu.SemaphoreType.DMA,
            pltpu.SemaphoreType.DMA,
        ],
        compiler_params=pltpu.CompilerParams(needs_layout_passes=False),
    )
    def k1(tt, out, tin0, tin1, rout0, rout1, tin_t, rout_t,
           isem0, isem1, osem0, osem1):
        wid = lax.axis_index("s") * info.num_cores + lax.axis_index("c")
        tin = (tin0, tin1)
        rout = (rout0, rout1)
        isem = (isem0, isem1)
        osem = (osem0, osem1)
        iot = lax.iota(jnp.int32, 16)
        n_valid = (_FULL_CHUNKS - wid + nw - 1) // nw

        def in_slice(c):
            return tt.at[:, pl.ds(c * _W, _W)]

        def out_slice(c):
            return out.at[pl.ds(c * (_W * _N_EMBD), _W * _N_EMBD)]

        def start_in(j, b):
            c = wid + j * nw

            @pl.when(c < _FULL_CHUNKS)
            def _():
                pltpu.async_copy(in_slice(c), tin[b].at[:, pl.ds(0, _W)],
                                 isem[b])

        def transpose_chunk(src, dst, ncols):
            def vbody(v, carry):
                vcol = jnp.full((16,), 0, jnp.int32) + v
                lo = plsc.load_gather(src, [iot, vcol])
                hi = plsc.load_gather(src, [iot + 16, vcol])
                dst[pl.ds(v * _N_EMBD, 16)] = lo
                dst[pl.ds(v * _N_EMBD + 16, 16)] = hi
                return carry

            lax.fori_loop(0, ncols, vbody, 0, unroll=8)

        start_in(0, 0)
        start_in(1, 1)

        def body2(j2, carry):
            for b in range(2):
                j = j2 * 2 + b
                c = wid + j * nw

                @pl.when(c < _FULL_CHUNKS)
                def _():
                    @pl.when(j2 >= 1)
                    def _():
                        pltpu.make_async_copy(
                            rout[b], out_slice(c), osem[b]).wait()

                    pltpu.make_async_copy(
                        in_slice(c), tin[b].at[:, pl.ds(0, _W)],
                        isem[b]).wait()
                    transpose_chunk(tin[b], rout[b], _W)
                    pltpu.async_copy(rout[b], out_slice(c), osem[b])
                    start_in(j + 2, b)

            return carry

        lax.fori_loop(0, max_j2, body2, 0)

        @pl.when(n_valid >= 1)
        def _drain0():
            pltpu.make_async_copy(rout[0], out_slice(0), osem[0]).wait()

        @pl.when(n_valid >= 2)
        def _drain1():
            pltpu.make_async_copy(rout[1], out_slice(0), osem[1]).wait()

        @pl.when(wid == nw - 1)
        def _tail():
            base = _FULL_CHUNKS * _W
            pltpu.sync_copy(tt.at[:, pl.ds(base, _TAIL)], tin_t)
            transpose_chunk(tin_t, rout_t, _TAIL)
            pltpu.sync_copy(
                rout_t,
                out.at[pl.ds(base * _N_EMBD, _TAIL * _N_EMBD)])

    return k1


@functools.cache
def _make_gather(B, D):
    info = plsc.get_sparse_core_info()
    nw = info.num_cores * info.num_subcores
    b_per_w = B // nw
    n_chunks = b_per_w // _CHUNK
    assert b_per_w % _CHUNK == 0
    mesh = plsc.VectorSubcoreMesh(core_axis_name="c", subcore_axis_name="s")

    @functools.partial(
        pl.kernel,
        mesh=mesh,
        out_type=jax.ShapeDtypeStruct((B, D), jnp.float32),
        scratch_types=[
            pltpu.VMEM((_CHUNK,), jnp.int32),
            pltpu.VMEM((_CHUNK, D), jnp.float32),
            pltpu.SemaphoreType.DMA,
        ],
        compiler_params=pltpu.CompilerParams(use_tc_tiling_on_sc=False),
    )
    def k2(idx_hbm, table_hbm, out_hbm, idx_v, rows_v, sem):
        wid = lax.axis_index("s") * info.num_cores + lax.axis_index("c")
        base = wid * b_per_w

        def body(j, carry):
            off = base + j * _CHUNK
            pltpu.sync_copy(idx_hbm.at[pl.ds(off, _CHUNK)], idx_v)
            pltpu.async_copy(table_hbm.at[idx_v], rows_v, sem).wait()
            pltpu.sync_copy(rows_v, out_hbm.at[pl.ds(off, _CHUNK)])
            return carry

        lax.fori_loop(0, n_chunks, body, 0)

    return k2


def kernel(inputs, wte):
    s0, s1 = inputs.shape
    idx = inputs.reshape(s0 * s1).astype(jnp.int32)
    table_lin = _make_detranspose()(wte.T).reshape(_N_VOCAB, _N_EMBD)
    gathered = _make_gather(s0 * s1, _N_EMBD)(idx, table_lin)
    return (gathered.reshape(s0, s1, _N_EMBD), wte)


# speedup vs baseline: 2.5654x; 1.2823x over previous
import functools

import jax
import jax.numpy as jnp
from jax import lax
from jax.experimental import pallas as pl
from jax.experimental.pallas import tpu as pltpu
from jax.experimental.pallas import tpu_sc as plsc

_N_VOCAB = 1000000
_N_EMBD = 32
_PITCH = _N_EMBD + 1
_CHUNK = 2048

_W = 512
_FULL_CHUNKS = _N_VOCAB // _W
_TAIL = _N_VOCAB - _FULL_CHUNKS * _W


@functools.cache
def _make_detranspose():
    info = plsc.get_sparse_core_info()
    nw = info.num_cores * info.num_subcores
    max_j2 = (-(-_FULL_CHUNKS // nw) + 1) // 2
    mesh = plsc.VectorSubcoreMesh(core_axis_name="c", subcore_axis_name="s")

    @functools.partial(
        pl.kernel,
        mesh=mesh,
        out_type=jax.ShapeDtypeStruct((_N_VOCAB * _N_EMBD,), jnp.float32),
        scratch_types=[
            pltpu.VMEM((_N_EMBD, _W), jnp.float32),
            pltpu.VMEM((_N_EMBD, _W), jnp.float32),
            pltpu.VMEM((_W * _PITCH,), jnp.float32),
            pltpu.VMEM((_W * _N_EMBD,), jnp.float32),
            pltpu.VMEM((_W * _N_EMBD,), jnp.float32),
            pltpu.VMEM((_N_EMBD, _TAIL), jnp.float32),
            pltpu.VMEM((_TAIL * _N_EMBD,), jnp.float32),
            pltpu.SemaphoreType.DMA,
            pltpu.SemaphoreType.DMA,
            pltpu.SemaphoreType.DMA,
            pltpu.SemaphoreType.DMA,
        ],
        compiler_params=pltpu.CompilerParams(needs_layout_passes=False),
    )
    def k1(tt, out, tin0, tin1, rpad, rout0, rout1, tin_t, rout_t,
           isem0, isem1, osem0, osem1):
        wid = lax.axis_index("s") * info.num_cores + lax.axis_index("c")
        tin = (tin0, tin1)
        rout = (rout0, rout1)
        isem = (isem0, isem1)
        osem = (osem0, osem1)
        iotp = lax.iota(jnp.int32, 16) * _PITCH
        n_valid = (_FULL_CHUNKS - wid + nw - 1) // nw

        def in_slice(c):
            return tt.at[:, pl.ds(c * _W, _W)]

        def out_slice(c):
            return out.at[pl.ds(c * (_W * _N_EMBD), _W * _N_EMBD)]

        def start_in(j, b):
            c = wid + j * nw

            @pl.when(c < _FULL_CHUNKS)
            def _():
                pltpu.async_copy(in_slice(c), tin[b], isem[b])

        def transpose_chunk(src, dst, ncols):
            def kbody(k, carry):
                base = iotp + k * 16 * _PITCH

                def ebody(e, carry2):
                    val = src[e, pl.ds(k * 16, 16)]
                    plsc.store_scatter(rpad, [base + e], val)
                    return carry2

                lax.fori_loop(0, _N_EMBD, ebody, 0, unroll=8)
                return carry

            lax.fori_loop(0, ncols // 16, kbody, 0)

            def vbody(v, carry):
                dst[pl.ds(v * _N_EMBD, 16)] = rpad[pl.ds(v * _PITCH, 16)]
                dst[pl.ds(v * _N_EMBD + 16, 16)] = (
                    rpad[pl.ds(v * _PITCH + 16, 16)])
                return carry

            lax.fori_loop(0, ncols, vbody, 0, unroll=8)

        start_in(0, 0)
        start_in(1, 1)

        def body2(j2, carry):
            for b in range(2):
                j = j2 * 2 + b
                c = wid + j * nw

                @pl.when(c < _FULL_CHUNKS)
                def _():
                    @pl.when(j2 >= 1)
                    def _():
                        pltpu.make_async_copy(
                            rout[b], out_slice(c), osem[b]).wait()

                    pltpu.make_async_copy(in_slice(c), tin[b], isem[b]).wait()
                    transpose_chunk(tin[b], rout[b], _W)
                    pltpu.async_copy(rout[b], out_slice(c), osem[b])
                    start_in(j + 2, b)

            return carry

        lax.fori_loop(0, max_j2, body2, 0)

        @pl.when(n_valid >= 1)
        def _drain0():
            pltpu.make_async_copy(rout[0], out_slice(0), osem[0]).wait()

        @pl.when(n_valid >= 2)
        def _drain1():
            pltpu.make_async_copy(rout[1], out_slice(0), osem[1]).wait()

        @pl.when(wid == nw - 1)
        def _tail():
            base = _FULL_CHUNKS * _W
            pltpu.sync_copy(tt.at[:, pl.ds(base, _TAIL)], tin_t)
            transpose_chunk(tin_t, rout_t, _TAIL)
            pltpu.sync_copy(
                rout_t,
                out.at[pl.ds(base * _N_EMBD, _TAIL * _N_EMBD)])

    return k1


@functools.cache
def _make_gather(B, D):
    info = plsc.get_sparse_core_info()
    nw = info.num_cores * info.num_subcores
    b_per_w = B // nw
    n_chunks = b_per_w // _CHUNK
    assert b_per_w % _CHUNK == 0
    mesh = plsc.VectorSubcoreMesh(core_axis_name="c", subcore_axis_name="s")

    @functools.partial(
        pl.kernel,
        mesh=mesh,
        out_type=jax.ShapeDtypeStruct((B, D), jnp.float32),
        scratch_types=[
            pltpu.VMEM((_CHUNK,), jnp.int32),
            pltpu.VMEM((_CHUNK, D), jnp.float32),
            pltpu.SemaphoreType.DMA,
        ],
        compiler_params=pltpu.CompilerParams(use_tc_tiling_on_sc=False),
    )
    def k2(idx_hbm, table_hbm, out_hbm, idx_v, rows_v, sem):
        wid = lax.axis_index("s") * info.num_cores + lax.axis_index("c")
        base = wid * b_per_w

        def body(j, carry):
            off = base + j * _CHUNK
            pltpu.sync_copy(idx_hbm.at[pl.ds(off, _CHUNK)], idx_v)
            pltpu.async_copy(table_hbm.at[idx_v], rows_v, sem).wait()
            pltpu.sync_copy(rows_v, out_hbm.at[pl.ds(off, _CHUNK)])
            return carry

        lax.fori_loop(0, n_chunks, body, 0)

    return k2


def kernel(inputs, wte):
    s0, s1 = inputs.shape
    idx = inputs.reshape(s0 * s1).astype(jnp.int32)
    table_lin = _make_detranspose()(wte.T).reshape(_N_VOCAB, _N_EMBD)
    gathered = _make_gather(s0 * s1, _N_EMBD)(idx, table_lin)
    return (gathered.reshape(s0, s1, _N_EMBD), wte)


# speedup vs baseline: 2.5909x; 1.0099x over previous
import functools

import jax
import jax.numpy as jnp
from jax import lax
from jax.experimental import pallas as pl
from jax.experimental.pallas import tpu as pltpu
from jax.experimental.pallas import tpu_sc as plsc

_N_VOCAB = 1000000
_N_EMBD = 32
_PITCH = _N_EMBD + 1
_CHUNK = 2048

_W = 512
_FULL_CHUNKS = _N_VOCAB // _W
_TAIL = _N_VOCAB - _FULL_CHUNKS * _W


@functools.cache
def _make_detranspose():
    info = plsc.get_sparse_core_info()
    nw = info.num_cores * info.num_subcores
    max_j2 = (-(-_FULL_CHUNKS // nw) + 1) // 2
    mesh = plsc.VectorSubcoreMesh(core_axis_name="c", subcore_axis_name="s")

    @functools.partial(
        pl.kernel,
        mesh=mesh,
        out_type=jax.ShapeDtypeStruct((_N_VOCAB * _N_EMBD,), jnp.float32),
        scratch_types=[
            pltpu.VMEM((_N_EMBD, _W), jnp.float32),
            pltpu.VMEM((_N_EMBD, _W), jnp.float32),
            pltpu.VMEM((_W * _PITCH,), jnp.float32),
            pltpu.VMEM((_W * _N_EMBD,), jnp.float32),
            pltpu.VMEM((_W * _N_EMBD,), jnp.float32),
            pltpu.VMEM((_N_EMBD, _TAIL), jnp.float32),
            pltpu.VMEM((_TAIL * _N_EMBD,), jnp.float32),
            pltpu.SemaphoreType.DMA,
            pltpu.SemaphoreType.DMA,
            pltpu.SemaphoreType.DMA,
            pltpu.SemaphoreType.DMA,
        ],
        compiler_params=pltpu.CompilerParams(needs_layout_passes=False),
    )
    def k1(tt, out, tin0, tin1, rpad, rout0, rout1, tin_t, rout_t,
           isem0, isem1, osem0, osem1):
        wid = lax.axis_index("s") * info.num_cores + lax.axis_index("c")
        tin = (tin0, tin1)
        rout = (rout0, rout1)
        isem = (isem0, isem1)
        osem = (osem0, osem1)
        iotp = lax.iota(jnp.int32, 16) * _PITCH
        n_valid = (_FULL_CHUNKS - wid + nw - 1) // nw

        def in_slice(c):
            return tt.at[:, pl.ds(c * _W, _W)]

        def out_slice(c):
            return out.at[pl.ds(c * (_W * _N_EMBD), _W * _N_EMBD)]

        def start_in(j, b):
            c = wid + j * nw

            @pl.when(c < _FULL_CHUNKS)
            def _():
                pltpu.async_copy(in_slice(c), tin[b], isem[b])

        def transpose_chunk(src, dst, ncols):
            def kbody(k, carry):
                voff, base = carry

                def ebody(e, base2):
                    val = src[e, pl.ds(voff, 16)]
                    plsc.store_scatter(rpad, [base2], val)
                    return base2 + 1

                lax.fori_loop(0, _N_EMBD, ebody, base, unroll=8)
                return (voff + 16, base + 16 * _PITCH)

            lax.fori_loop(0, ncols // 16, kbody, (0, iotp))

            def vbody(v, carry):
                o32, o33 = carry
                dst[pl.ds(o32, 16)] = rpad[pl.ds(o33, 16)]
                dst[pl.ds(o32 + 16, 16)] = rpad[pl.ds(o33 + 16, 16)]
                return (o32 + _N_EMBD, o33 + _PITCH)

            lax.fori_loop(0, ncols, vbody, (0, 0), unroll=8)

        start_in(0, 0)
        start_in(1, 1)

        def body2(j2, carry):
            for b in range(2):
                j = j2 * 2 + b
                c = wid + j * nw

                @pl.when(c < _FULL_CHUNKS)
                def _():
                    @pl.when(j2 >= 1)
                    def _():
                        pltpu.make_async_copy(
                            rout[b], out_slice(c), osem[b]).wait()

                    pltpu.make_async_copy(in_slice(c), tin[b], isem[b]).wait()
                    transpose_chunk(tin[b], rout[b], _W)
                    pltpu.async_copy(rout[b], out_slice(c), osem[b])
                    start_in(j + 2, b)

            return carry

        lax.fori_loop(0, max_j2, body2, 0)

        @pl.when(n_valid >= 1)
        def _drain0():
            pltpu.make_async_copy(rout[0], out_slice(0), osem[0]).wait()

        @pl.when(n_valid >= 2)
        def _drain1():
            pltpu.make_async_copy(rout[1], out_slice(0), osem[1]).wait()

        @pl.when(wid == nw - 1)
        def _tail():
            base = _FULL_CHUNKS * _W
            pltpu.sync_copy(tt.at[:, pl.ds(base, _TAIL)], tin_t)
            transpose_chunk(tin_t, rout_t, _TAIL)
            pltpu.sync_copy(
                rout_t,
                out.at[pl.ds(base * _N_EMBD, _TAIL * _N_EMBD)])

    return k1


@functools.cache
def _make_gather(B, D):
    info = plsc.get_sparse_core_info()
    nw = info.num_cores * info.num_subcores
    b_per_w = B // nw
    n_chunks = b_per_w // _CHUNK
    assert b_per_w % _CHUNK == 0
    mesh = plsc.VectorSubcoreMesh(core_axis_name="c", subcore_axis_name="s")

    @functools.partial(
        pl.kernel,
        mesh=mesh,
        out_type=jax.ShapeDtypeStruct((B, D), jnp.float32),
        scratch_types=[
            pltpu.VMEM((_CHUNK,), jnp.int32),
            pltpu.VMEM((_CHUNK, D), jnp.float32),
            pltpu.SemaphoreType.DMA,
        ],
        compiler_params=pltpu.CompilerParams(use_tc_tiling_on_sc=False),
    )
    def k2(idx_hbm, table_hbm, out_hbm, idx_v, rows_v, sem):
        wid = lax.axis_index("s") * info.num_cores + lax.axis_index("c")
        base = wid * b_per_w

        def body(j, carry):
            off = base + j * _CHUNK
            pltpu.sync_copy(idx_hbm.at[pl.ds(off, _CHUNK)], idx_v)
            pltpu.async_copy(table_hbm.at[idx_v], rows_v, sem).wait()
            pltpu.sync_copy(rows_v, out_hbm.at[pl.ds(off, _CHUNK)])
            return carry

        lax.fori_loop(0, n_chunks, body, 0)

    return k2


def kernel(inputs, wte):
    s0, s1 = inputs.shape
    idx = inputs.reshape(s0 * s1).astype(jnp.int32)
    table_lin = _make_detranspose()(wte.T).reshape(_N_VOCAB, _N_EMBD)
    gathered = _make_gather(s0 * s1, _N_EMBD)(idx, table_lin)
    return (gathered.reshape(s0, s1, _N_EMBD), wte)


# speedup vs baseline: 2.8807x; 1.1119x over previous
import functools

import jax
import jax.numpy as jnp
from jax import lax
from jax.experimental import pallas as pl
from jax.experimental.pallas import tpu as pltpu
from jax.experimental.pallas import tpu_sc as plsc

_N_VOCAB = 1000000
_N_EMBD = 32
_PITCH = _N_EMBD + 1
_CHUNK = 2048

_W = 512
_FULL_CHUNKS = _N_VOCAB // _W
_TAIL = _N_VOCAB - _FULL_CHUNKS * _W


@functools.cache
def _make_detranspose():
    info = plsc.get_sparse_core_info()
    nw = info.num_cores * info.num_subcores
    max_j2 = (-(-_FULL_CHUNKS // nw) + 1) // 2
    mesh = plsc.VectorSubcoreMesh(core_axis_name="c", subcore_axis_name="s")

    @functools.partial(
        pl.kernel,
        mesh=mesh,
        out_type=[
            jax.ShapeDtypeStruct((_N_VOCAB * _N_EMBD,), jnp.float32),
            jax.ShapeDtypeStruct((_N_EMBD, _N_VOCAB), jnp.float32),
        ],
        scratch_types=[
            pltpu.VMEM((_N_EMBD, _W), jnp.float32),
            pltpu.VMEM((_N_EMBD, _W), jnp.float32),
            pltpu.VMEM((_W * _PITCH,), jnp.float32),
            pltpu.VMEM((_W * _N_EMBD,), jnp.float32),
            pltpu.VMEM((_W * _N_EMBD,), jnp.float32),
            pltpu.VMEM((_N_EMBD, _TAIL), jnp.float32),
            pltpu.VMEM((_TAIL * _N_EMBD,), jnp.float32),
            pltpu.SemaphoreType.DMA,
            pltpu.SemaphoreType.DMA,
            pltpu.SemaphoreType.DMA,
            pltpu.SemaphoreType.DMA,
            pltpu.SemaphoreType.DMA,
            pltpu.SemaphoreType.DMA,
        ],
        compiler_params=pltpu.CompilerParams(needs_layout_passes=False),
    )
    def k1(tt, out, out2, tin0, tin1, rpad, rout0, rout1, tin_t, rout_t,
           isem0, isem1, osem0, osem1, psem0, psem1):
        wid = lax.axis_index("s") * info.num_cores + lax.axis_index("c")
        tin = (tin0, tin1)
        rout = (rout0, rout1)
        isem = (isem0, isem1)
        osem = (osem0, osem1)
        psem = (psem0, psem1)
        iotp = lax.iota(jnp.int32, 16) * _PITCH
        n_valid = (_FULL_CHUNKS - wid + nw - 1) // nw

        def in_slice(c):
            return tt.at[:, pl.ds(c * _W, _W)]

        def out_slice(c):
            return out.at[pl.ds(c * (_W * _N_EMBD), _W * _N_EMBD)]

        def out2_slice(c):
            return out2.at[:, pl.ds(c * _W, _W)]

        def start_in(j, b):
            c = wid + j * nw

            @pl.when(c < _FULL_CHUNKS)
            def _():
                pltpu.async_copy(in_slice(c), tin[b], isem[b])

        def transpose_chunk(src, dst, ncols):
            def kbody(k, carry):
                voff, base = carry

                def ebody(e, base2):
                    val = src[e, pl.ds(voff, 16)]
                    plsc.store_scatter(rpad, [base2], val)
                    return base2 + 1

                lax.fori_loop(0, _N_EMBD, ebody, base, unroll=8)
                return (voff + 16, base + 16 * _PITCH)

            lax.fori_loop(0, ncols // 16, kbody, (0, iotp))

            def vbody(v, carry):
                o32, o33 = carry
                dst[pl.ds(o32, 16)] = rpad[pl.ds(o33, 16)]
                dst[pl.ds(o32 + 16, 16)] = rpad[pl.ds(o33 + 16, 16)]
                return (o32 + _N_EMBD, o33 + _PITCH)

            lax.fori_loop(0, ncols, vbody, (0, 0), unroll=8)

        start_in(0, 0)
        start_in(1, 1)

        def body2(j2, carry):
            for b in range(2):
                j = j2 * 2 + b
                c = wid + j * nw

                @pl.when(c < _FULL_CHUNKS)
                def _():
                    @pl.when(j2 >= 1)
                    def _():
                        pltpu.make_async_copy(
                            rout[b], out_slice(c), osem[b]).wait()

                    pltpu.make_async_copy(in_slice(c), tin[b], isem[b]).wait()
                    pltpu.async_copy(tin[b], out2_slice(c), psem[b])
                    transpose_chunk(tin[b], rout[b], _W)
                    pltpu.async_copy(rout[b], out_slice(c), osem[b])
                    pltpu.make_async_copy(tin[b], out2_slice(c),
                                          psem[b]).wait()
                    start_in(j + 2, b)

            return carry

        lax.fori_loop(0, max_j2, body2, 0)

        @pl.when(n_valid >= 1)
        def _drain0():
            pltpu.make_async_copy(rout[0], out_slice(0), osem[0]).wait()

        @pl.when(n_valid >= 2)
        def _drain1():
            pltpu.make_async_copy(rout[1], out_slice(0), osem[1]).wait()

        @pl.when(wid == nw - 1)
        def _tail():
            base = _FULL_CHUNKS * _W
            pltpu.sync_copy(tt.at[:, pl.ds(base, _TAIL)], tin_t)
            pltpu.sync_copy(tin_t, out2.at[:, pl.ds(base, _TAIL)])
            transpose_chunk(tin_t, rout_t, _TAIL)
            pltpu.sync_copy(
                rout_t,
                out.at[pl.ds(base * _N_EMBD, _TAIL * _N_EMBD)])

    return k1


@functools.cache
def _make_gather(B, D):
    info = plsc.get_sparse_core_info()
    nw = info.num_cores * info.num_subcores
    b_per_w = B // nw
    n_chunks = b_per_w // _CHUNK
    assert b_per_w % _CHUNK == 0
    mesh = plsc.VectorSubcoreMesh(core_axis_name="c", subcore_axis_name="s")

    @functools.partial(
        pl.kernel,
        mesh=mesh,
        out_type=jax.ShapeDtypeStruct((B, D), jnp.float32),
        scratch_types=[
            pltpu.VMEM((_CHUNK,), jnp.int32),
            pltpu.VMEM((_CHUNK, D), jnp.float32),
            pltpu.SemaphoreType.DMA,
        ],
        compiler_params=pltpu.CompilerParams(use_tc_tiling_on_sc=False),
    )
    def k2(idx_hbm, table_hbm, out_hbm, idx_v, rows_v, sem):
        wid = lax.axis_index("s") * info.num_cores + lax.axis_index("c")
        base = wid * b_per_w

        def body(j, carry):
            off = base + j * _CHUNK
            pltpu.sync_copy(idx_hbm.at[pl.ds(off, _CHUNK)], idx_v)
            pltpu.async_copy(table_hbm.at[idx_v], rows_v, sem).wait()
            pltpu.sync_copy(rows_v, out_hbm.at[pl.ds(off, _CHUNK)])
            return carry

        lax.fori_loop(0, n_chunks, body, 0)

    return k2


def kernel(inputs, wte):
    s0, s1 = inputs.shape
    idx = inputs.reshape(s0 * s1).astype(jnp.int32)
    table_flat, wte_t = _make_detranspose()(wte.T)
    table_lin = table_flat.reshape(_N_VOCAB, _N_EMBD)
    gathered = _make_gather(s0 * s1, _N_EMBD)(idx, table_lin)
    return (gathered.reshape(s0, s1, _N_EMBD), wte_t.T)


# speedup vs baseline: 2.8915x; 1.0038x over previous
import functools

import jax
import jax.numpy as jnp
from jax import lax
from jax.experimental import pallas as pl
from jax.experimental.pallas import tpu as pltpu
from jax.experimental.pallas import tpu_sc as plsc

_N_VOCAB = 1000000
_N_EMBD = 32
_PITCH = _N_EMBD + 1
_CHUNK = 2048

_W = 512
_FULL_CHUNKS = _N_VOCAB // _W
_TAIL = _N_VOCAB - _FULL_CHUNKS * _W


@functools.cache
def _make_detranspose():
    info = plsc.get_sparse_core_info()
    nw = info.num_cores * info.num_subcores
    max_j2 = (-(-_FULL_CHUNKS // nw) + 1) // 2
    mesh = plsc.VectorSubcoreMesh(core_axis_name="c", subcore_axis_name="s")

    @functools.partial(
        pl.kernel,
        mesh=mesh,
        out_type=[
            jax.ShapeDtypeStruct((_N_VOCAB * _N_EMBD,), jnp.float32),
            jax.ShapeDtypeStruct((_N_EMBD, _N_VOCAB), jnp.float32),
        ],
        scratch_types=[
            pltpu.VMEM((_N_EMBD, _W), jnp.float32),
            pltpu.VMEM((_N_EMBD, _W), jnp.float32),
            pltpu.VMEM((_W * _PITCH,), jnp.float32),
            pltpu.VMEM((_W * _N_EMBD,), jnp.float32),
            pltpu.VMEM((_W * _N_EMBD,), jnp.float32),
            pltpu.VMEM((_N_EMBD, _TAIL), jnp.float32),
            pltpu.VMEM((_TAIL * _N_EMBD,), jnp.float32),
            pltpu.SemaphoreType.DMA,
            pltpu.SemaphoreType.DMA,
            pltpu.SemaphoreType.DMA,
            pltpu.SemaphoreType.DMA,
            pltpu.SemaphoreType.DMA,
            pltpu.SemaphoreType.DMA,
        ],
        compiler_params=pltpu.CompilerParams(needs_layout_passes=False),
    )
    def k1(tt, out, out2, tin0, tin1, rpad, rout0, rout1, tin_t, rout_t,
           isem0, isem1, osem0, osem1, psem0, psem1):
        wid = lax.axis_index("s") * info.num_cores + lax.axis_index("c")
        tin = (tin0, tin1)
        rout = (rout0, rout1)
        isem = (isem0, isem1)
        osem = (osem0, osem1)
        psem = (psem0, psem1)
        iotp = lax.iota(jnp.int32, 16) * _PITCH
        n_valid = (_FULL_CHUNKS - wid + nw - 1) // nw

        def in_slice(c):
            return tt.at[:, pl.ds(c * _W, _W)]

        def out_slice(c):
            return out.at[pl.ds(c * (_W * _N_EMBD), _W * _N_EMBD)]

        def out2_slice(c):
            return out2.at[:, pl.ds(c * _W, _W)]

        def start_in(j, b):
            c = wid + j * nw

            @pl.when(c < _FULL_CHUNKS)
            def _():
                pltpu.async_copy(in_slice(c), tin[b], isem[b])

        def transpose_chunk(src, dst, ncols):
            def kbody(k, carry):
                voff, base = carry

                def ebody(e, base2):
                    val = src[e, pl.ds(voff, 16)]
                    plsc.store_scatter(rpad, [base2], val)
                    return base2 + 1

                lax.fori_loop(0, _N_EMBD, ebody, base, unroll=16)
                return (voff + 16, base + 16 * _PITCH)

            lax.fori_loop(0, ncols // 16, kbody, (0, iotp))

            def vbody(v, carry):
                o32, o33 = carry
                dst[pl.ds(o32, 16)] = rpad[pl.ds(o33, 16)]
                dst[pl.ds(o32 + 16, 16)] = rpad[pl.ds(o33 + 16, 16)]
                return (o32 + _N_EMBD, o33 + _PITCH)

            lax.fori_loop(0, ncols, vbody, (0, 0), unroll=16)

        start_in(0, 0)
        start_in(1, 1)

        def body2(j2, carry):
            for b in range(2):
                j = j2 * 2 + b
                c = wid + j * nw

                @pl.when(c < _FULL_CHUNKS)
                def _():
                    @pl.when(j2 >= 1)
                    def _():
                        pltpu.make_async_copy(
                            rout[b], out_slice(c), osem[b]).wait()

                    pltpu.make_async_copy(in_slice(c), tin[b], isem[b]).wait()
                    pltpu.async_copy(tin[b], out2_slice(c), psem[b])
                    transpose_chunk(tin[b], rout[b], _W)
                    pltpu.async_copy(rout[b], out_slice(c), osem[b])
                    pltpu.make_async_copy(tin[b], out2_slice(c),
                                          psem[b]).wait()
                    start_in(j + 2, b)

            return carry

        lax.fori_loop(0, max_j2, body2, 0)

        @pl.when(n_valid >= 1)
        def _drain0():
            pltpu.make_async_copy(rout[0], out_slice(0), osem[0]).wait()

        @pl.when(n_valid >= 2)
        def _drain1():
            pltpu.make_async_copy(rout[1], out_slice(0), osem[1]).wait()

        @pl.when(wid == nw - 1)
        def _tail():
            base = _FULL_CHUNKS * _W
            pltpu.sync_copy(tt.at[:, pl.ds(base, _TAIL)], tin_t)
            pltpu.sync_copy(tin_t, out2.at[:, pl.ds(base, _TAIL)])
            transpose_chunk(tin_t, rout_t, _TAIL)
            pltpu.sync_copy(
                rout_t,
                out.at[pl.ds(base * _N_EMBD, _TAIL * _N_EMBD)])

    return k1


@functools.cache
def _make_gather(B, D):
    info = plsc.get_sparse_core_info()
    nw = info.num_cores * info.num_subcores
    b_per_w = B // nw
    n_chunks = b_per_w // _CHUNK
    assert b_per_w % _CHUNK == 0
    mesh = plsc.VectorSubcoreMesh(core_axis_name="c", subcore_axis_name="s")

    @functools.partial(
        pl.kernel,
        mesh=mesh,
        out_type=jax.ShapeDtypeStruct((B, D), jnp.float32),
        scratch_types=[
            pltpu.VMEM((_CHUNK,), jnp.int32),
            pltpu.VMEM((_CHUNK, D), jnp.float32),
            pltpu.SemaphoreType.DMA,
        ],
        compiler_params=pltpu.CompilerParams(use_tc_tiling_on_sc=False),
    )
    def k2(idx_hbm, table_hbm, out_hbm, idx_v, rows_v, sem):
        wid = lax.axis_index("s") * info.num_cores + lax.axis_index("c")
        base = wid * b_per_w

        def body(j, carry):
            off = base + j * _CHUNK
            pltpu.sync_copy(idx_hbm.at[pl.ds(off, _CHUNK)], idx_v)
            pltpu.async_copy(table_hbm.at[idx_v], rows_v, sem).wait()
            pltpu.sync_copy(rows_v, out_hbm.at[pl.ds(off, _CHUNK)])
            return carry

        lax.fori_loop(0, n_chunks, body, 0)

    return k2


def kernel(inputs, wte):
    s0, s1 = inputs.shape
    idx = inputs.reshape(s0 * s1).astype(jnp.int32)
    table_flat, wte_t = _make_detranspose()(wte.T)
    table_lin = table_flat.reshape(_N_VOCAB, _N_EMBD)
    gathered = _make_gather(s0 * s1, _N_EMBD)(idx, table_lin)
    return (gathered.reshape(s0, s1, _N_EMBD), wte_t.T)
